# initial kernel scaffold (unmeasured)
import jax
import jax.numpy as jnp
from jax import lax
from jax.experimental import pallas as pl
from jax.experimental.pallas import tpu as pltpu


def kernel(x, dy):
    k, m = x.shape
    _, f = dy.shape
    half_m = m // 2

    def body(x_ref, dy_ref, out_ref, send_buf, recv_buf, send_sem, recv_sem):
        my_x = lax.axis_index("x")
        my_y = lax.axis_index("y")

        barrier_sem = pltpu.get_barrier_semaphore()
        pl.semaphore_signal(
            barrier_sem,
            inc=1,
            device_id=(my_x, 1 - my_y),
            device_id_type=pl.DeviceIdType.MESH,
        )
        pl.semaphore_wait(barrier_sem, 1)

        p = lax.dot_general(
            x_ref[...],
            dy_ref[...],
            dimension_numbers=(((0,), (0,)), ((), ())),
            preferred_element_type=jnp.float32,
        )

        send_buf[...] = lax.dynamic_slice_in_dim(
            p, (1 - my_y) * half_m, half_m, axis=0
        )

        rdma = pltpu.make_async_remote_copy(
            src_ref=send_buf,
            dst_ref=recv_buf,
            send_sem=send_sem,
            recv_sem=recv_sem,
            device_id=(my_x, 1 - my_y),
            device_id_type=pl.DeviceIdType.MESH,
        )
        rdma.start()
        rdma.wait()

        out_ref[...] = (
            lax.dynamic_slice_in_dim(p, my_y * half_m, half_m, axis=0)
            + recv_buf[...]
        )

    return pl.pallas_call(
        body,
        out_shape=jax.ShapeDtypeStruct((half_m, f), jnp.float32),
        in_specs=[
            pl.BlockSpec(memory_space=pltpu.VMEM),
            pl.BlockSpec(memory_space=pltpu.VMEM),
        ],
        out_specs=pl.BlockSpec(memory_space=pltpu.VMEM),
        scratch_shapes=[
            pltpu.VMEM((half_m, f), jnp.float32),
            pltpu.VMEM((half_m, f), jnp.float32),
            pltpu.SemaphoreType.DMA,
            pltpu.SemaphoreType.DMA,
        ],
        compiler_params=pltpu.CompilerParams(collective_id=0),
    )(x, dy)


# baseline (device time: 120489 ns/iter reference)
import jax
import jax.numpy as jnp
from jax import lax
from jax.experimental import pallas as pl
from jax.experimental.pallas import tpu as pltpu


def kernel(x, dy):
    k, m = x.shape
    _, f = dy.shape
    half_m = m // 2

    def body(x_ref, dy_ref, out_ref, send_buf, recv_buf, send_sem, recv_sem):
        my_x = lax.axis_index("x")
        my_y = lax.axis_index("y")

        barrier_sem = pltpu.get_barrier_semaphore()
        pl.semaphore_signal(
            barrier_sem,
            inc=1,
            device_id=(my_x, 1 - my_y),
            device_id_type=pl.DeviceIdType.MESH,
        )
        pl.semaphore_wait(barrier_sem, 1)

        def dot_half(h):
            return lax.dot_general(
                x_ref[:, h * half_m : (h + 1) * half_m],
                dy_ref[...],
                dimension_numbers=(((0,), (0,)), ((), ())),
                preferred_element_type=jnp.float32,
            )

        p_lo = dot_half(0)
        p_hi = dot_half(1)
        is_lo = my_y == 0
        send_buf[...] = jnp.where(is_lo, p_hi, p_lo)

        rdma = pltpu.make_async_remote_copy(
            src_ref=send_buf,
            dst_ref=recv_buf,
            send_sem=send_sem,
            recv_sem=recv_sem,
            device_id=(my_x, 1 - my_y),
            device_id_type=pl.DeviceIdType.MESH,
        )
        rdma.start()
        rdma.wait()

        out_ref[...] = jnp.where(is_lo, p_lo, p_hi) + recv_buf[...]

    return pl.pallas_call(
        body,
        out_shape=jax.ShapeDtypeStruct((half_m, f), jnp.float32),
        in_specs=[
            pl.BlockSpec(memory_space=pltpu.VMEM),
            pl.BlockSpec(memory_space=pltpu.VMEM),
        ],
        out_specs=pl.BlockSpec(memory_space=pltpu.VMEM),
        scratch_shapes=[
            pltpu.VMEM((half_m, f), jnp.float32),
            pltpu.VMEM((half_m, f), jnp.float32),
            pltpu.SemaphoreType.DMA,
            pltpu.SemaphoreType.DMA,
        ],
        compiler_params=pltpu.CompilerParams(
            collective_id=0,
            vmem_limit_bytes=100 * 1024 * 1024,
        ),
    )(x, dy)


# device time: 84599 ns/iter; 1.4242x vs baseline; 1.4242x over previous
import jax
import jax.numpy as jnp
from jax import lax
from jax.experimental import pallas as pl
from jax.experimental.pallas import tpu as pltpu

N_CHUNKS = 16


def kernel(x, dy):
    k, m = x.shape
    _, f = dy.shape
    half_m = m // 2
    f2 = f // 2
    cw = f2 // N_CHUNKS

    def body(
        x_ref,
        dy_ref,
        out_ref,
        s_buf,
        yrecv,
        xrecv,
        ysend_sems,
        yrecv_sems,
        fsend_sems,
        xrecv_sems,
    ):
        my_x = lax.axis_index("x")
        my_y = lax.axis_index("y")
        y_nbr = (my_x, 1 - my_y)
        x_nbr = (1 - my_x, my_y)

        barrier_sem = pltpu.get_barrier_semaphore()
        for nbr in (y_nbr, x_nbr):
            pl.semaphore_signal(
                barrier_sem,
                inc=1,
                device_id=nbr,
                device_id_type=pl.DeviceIdType.MESH,
            )
        pl.semaphore_wait(barrier_sem, 2)

        is_lo_y = my_y == 0
        is_x0 = my_x == 0
        x_mine = jnp.where(is_lo_y, x_ref[:, :half_m], x_ref[:, half_m:])
        x_other = jnp.where(is_lo_y, x_ref[:, half_m:], x_ref[:, :half_m])

        y_rdmas = []
        for c in range(N_CHUNKS):
            lo, hi = c * cw, (c + 1) * cw
            s0 = lax.dot_general(
                x_other,
                dy_ref[:, lo:hi],
                dimension_numbers=(((0,), (0,)), ((), ())),
                preferred_element_type=jnp.float32,
            )
            s1 = lax.dot_general(
                x_other,
                dy_ref[:, f2 + lo : f2 + hi],
                dimension_numbers=(((0,), (0,)), ((), ())),
                preferred_element_type=jnp.float32,
            )
            s_buf[c] = jnp.where(is_x0, s0, s1)
            rdma = pltpu.make_async_remote_copy(
                src_ref=s_buf.at[c],
                dst_ref=yrecv.at[c],
                send_sem=ysend_sems.at[c],
                recv_sem=yrecv_sems.at[c],
                device_id=y_nbr,
                device_id_type=pl.DeviceIdType.MESH,
            )
            rdma.start()
            y_rdmas.append(rdma)

        l_val = lax.dot_general(
            x_mine,
            dy_ref[...],
            dimension_numbers=(((0,), (0,)), ((), ())),
            preferred_element_type=jnp.float32,
        )

        f_rdmas = []
        for c in range(N_CHUNKS):
            lo, hi = c * cw, (c + 1) * cw
            y_rdmas[c].wait_recv()
            fwd = pltpu.make_async_remote_copy(
                src_ref=yrecv.at[c],
                dst_ref=xrecv.at[c],
                send_sem=fsend_sems.at[c],
                recv_sem=xrecv_sems.at[c],
                device_id=x_nbr,
                device_id_type=pl.DeviceIdType.MESH,
            )
            fwd.start()
            f_rdmas.append(fwd)

            @pl.when(is_x0)
            def _():
                out_ref[:, lo:hi] = l_val[:, lo:hi] + yrecv[c]

            @pl.when(jnp.logical_not(is_x0))
            def _():
                out_ref[:, f2 + lo : f2 + hi] = (
                    l_val[:, f2 + lo : f2 + hi] + yrecv[c]
                )

        for c in range(N_CHUNKS):
            lo, hi = c * cw, (c + 1) * cw
            f_rdmas[c].wait_recv()

            @pl.when(is_x0)
            def _():
                out_ref[:, f2 + lo : f2 + hi] = (
                    l_val[:, f2 + lo : f2 + hi] + xrecv[c]
                )

            @pl.when(jnp.logical_not(is_x0))
            def _():
                out_ref[:, lo:hi] = l_val[:, lo:hi] + xrecv[c]

        for c in range(N_CHUNKS):
            y_rdmas[c].wait_send()
            f_rdmas[c].wait_send()

    return pl.pallas_call(
        body,
        out_shape=jax.ShapeDtypeStruct((half_m, f), jnp.float32),
        in_specs=[
            pl.BlockSpec(memory_space=pltpu.VMEM),
            pl.BlockSpec(memory_space=pltpu.VMEM),
        ],
        out_specs=pl.BlockSpec(memory_space=pltpu.VMEM),
        scratch_shapes=[
            pltpu.VMEM((N_CHUNKS, half_m, cw), jnp.float32),
            pltpu.VMEM((N_CHUNKS, half_m, cw), jnp.float32),
            pltpu.VMEM((N_CHUNKS, half_m, cw), jnp.float32),
            pltpu.SemaphoreType.DMA((N_CHUNKS,)),
            pltpu.SemaphoreType.DMA((N_CHUNKS,)),
            pltpu.SemaphoreType.DMA((N_CHUNKS,)),
            pltpu.SemaphoreType.DMA((N_CHUNKS,)),
        ],
        compiler_params=pltpu.CompilerParams(
            collective_id=0,
            vmem_limit_bytes=100 * 1024 * 1024,
        ),
    )(x, dy)


# device time: 71853 ns/iter; 1.6769x vs baseline; 1.1774x over previous
import jax
import jax.numpy as jnp
from jax import lax
from jax.experimental import pallas as pl
from jax.experimental.pallas import tpu as pltpu

N_CHUNKS = 16
LAG = 2


def kernel(x, dy):
    k, m = x.shape
    _, f = dy.shape
    half_m = m // 2
    f2 = f // 2
    cw = f2 // N_CHUNKS

    def body(
        x_ref,
        dy_ref,
        out_ref,
        s_buf,
        yrecv,
        xrecv,
        ysend_sems,
        yrecv_sems,
        fsend_sems,
        xrecv_sems,
    ):
        my_x = lax.axis_index("x")
        my_y = lax.axis_index("y")
        y_nbr = (my_x, 1 - my_y)
        x_nbr = (1 - my_x, my_y)

        barrier_sem = pltpu.get_barrier_semaphore()
        for nbr in (y_nbr, x_nbr):
            pl.semaphore_signal(
                barrier_sem,
                inc=1,
                device_id=nbr,
                device_id_type=pl.DeviceIdType.MESH,
            )
        pl.semaphore_wait(barrier_sem, 2)

        is_lo_y = my_y == 0
        is_x0 = my_x == 0
        x_mine = jnp.where(is_lo_y, x_ref[:, :half_m], x_ref[:, half_m:])
        x_other = jnp.where(is_lo_y, x_ref[:, half_m:], x_ref[:, :half_m])

        def dot(a, b):
            return lax.dot_general(
                a,
                b,
                dimension_numbers=(((0,), (0,)), ((), ())),
                preferred_element_type=jnp.float32,
            )

        y_rdmas = []
        f_rdmas = []

        def make_fwd(c):
            return pltpu.make_async_remote_copy(
                src_ref=yrecv.at[c],
                dst_ref=xrecv.at[c],
                send_sem=fsend_sems.at[c],
                recv_sem=xrecv_sems.at[c],
                device_id=x_nbr,
                device_id_type=pl.DeviceIdType.MESH,
            )

        def consume_ychunk(j):
            lo, hi = j * cw, (j + 1) * cw
            y_rdmas[j].wait_recv()
            fwd = make_fwd(j)
            fwd.start()
            f_rdmas.append(fwd)

            @pl.when(is_x0)
            def _():
                out_ref[:, lo:hi] = out_ref[:, lo:hi] + yrecv[j]

            @pl.when(jnp.logical_not(is_x0))
            def _():
                out_ref[:, f2 + lo : f2 + hi] = (
                    out_ref[:, f2 + lo : f2 + hi] + yrecv[j]
                )

        for c in range(N_CHUNKS):
            lo, hi = c * cw, (c + 1) * cw

            @pl.when(is_x0)
            def _():
                s_buf[c] = dot(x_other, dy_ref[:, lo:hi])

            @pl.when(jnp.logical_not(is_x0))
            def _():
                s_buf[c] = dot(x_other, dy_ref[:, f2 + lo : f2 + hi])

            rdma = pltpu.make_async_remote_copy(
                src_ref=s_buf.at[c],
                dst_ref=yrecv.at[c],
                send_sem=ysend_sems.at[c],
                recv_sem=yrecv_sems.at[c],
                device_id=y_nbr,
                device_id_type=pl.DeviceIdType.MESH,
            )
            rdma.start()
            y_rdmas.append(rdma)

            out_ref[:, lo:hi] = dot(x_mine, dy_ref[:, lo:hi])
            out_ref[:, f2 + lo : f2 + hi] = dot(
                x_mine, dy_ref[:, f2 + lo : f2 + hi]
            )

            if c >= LAG:
                consume_ychunk(c - LAG)

        for j in range(N_CHUNKS - LAG, N_CHUNKS):
            consume_ychunk(j)

        for j in range(N_CHUNKS):
            lo, hi = j * cw, (j + 1) * cw
            f_rdmas[j].wait_recv()

            @pl.when(is_x0)
            def _():
                out_ref[:, f2 + lo : f2 + hi] = (
                    out_ref[:, f2 + lo : f2 + hi] + xrecv[j]
                )

            @pl.when(jnp.logical_not(is_x0))
            def _():
                out_ref[:, lo:hi] = out_ref[:, lo:hi] + xrecv[j]

        for c in range(N_CHUNKS):
            y_rdmas[c].wait_send()
            f_rdmas[c].wait_send()

    return pl.pallas_call(
        body,
        out_shape=jax.ShapeDtypeStruct((half_m, f), jnp.float32),
        in_specs=[
            pl.BlockSpec(memory_space=pltpu.VMEM),
            pl.BlockSpec(memory_space=pltpu.VMEM),
        ],
        out_specs=pl.BlockSpec(memory_space=pltpu.VMEM),
        scratch_shapes=[
            pltpu.VMEM((N_CHUNKS, half_m, cw), jnp.float32),
            pltpu.VMEM((N_CHUNKS, half_m, cw), jnp.float32),
            pltpu.VMEM((N_CHUNKS, half_m, cw), jnp.float32),
            pltpu.SemaphoreType.DMA((N_CHUNKS,)),
            pltpu.SemaphoreType.DMA((N_CHUNKS,)),
            pltpu.SemaphoreType.DMA((N_CHUNKS,)),
            pltpu.SemaphoreType.DMA((N_CHUNKS,)),
        ],
        compiler_params=pltpu.CompilerParams(
            collective_id=0,
            vmem_limit_bytes=100 * 1024 * 1024,
        ),
    )(x, dy)
